# Initial kernel scaffold; baseline (speedup 1.0000x reference)
import jax
import jax.numpy as jnp
from jax.experimental import pallas as pl


def kernel(inputs, table, W1, b1, W2, b2):
    raise NotImplementedError("write your pallas kernel here")


# speedup vs baseline: 5.2558x; 5.2558x over previous
import functools

import jax
import jax.numpy as jnp
from jax import lax
from jax.experimental import pallas as pl
from jax.experimental.pallas import tpu as pltpu
from jax.experimental.pallas import tpu_sc as plsc

HASH_BIN = 100000
EMB_DIM = 32
H1 = 32
H2 = 64
BATCH = 16384
FIELDS = 26

TOTAL = BATCH * FIELDS
NW = 32
ROWS_PER_W = TOTAL // NW
CHUNK = 128
NCHUNK = ROWS_PER_W // CHUNK

ROW_BLOCK = 2000
N_BLOCKS = HASH_BIN // ROW_BLOCK


def _mlp_body(t_ref, w1_ref, b1_ref, w2_ref, b2_ref, o_ref):
    h = jnp.dot(t_ref[...], w1_ref[...], preferred_element_type=jnp.float32)
    h = jnp.maximum(h + b1_ref[...], 0.0)
    o = jnp.dot(h, w2_ref[...], preferred_element_type=jnp.float32)
    o_ref[...] = jnp.maximum(o + b2_ref[...], 0.0)


def _table_mlp(table, W1, b1, W2, b2):
    return pl.pallas_call(
        _mlp_body,
        grid=(N_BLOCKS,),
        in_specs=[
            pl.BlockSpec((ROW_BLOCK, EMB_DIM), lambda i: (i, 0)),
            pl.BlockSpec((EMB_DIM, H1), lambda i: (0, 0)),
            pl.BlockSpec((1, H1), lambda i: (0, 0)),
            pl.BlockSpec((H1, H2), lambda i: (0, 0)),
            pl.BlockSpec((1, H2), lambda i: (0, 0)),
        ],
        out_specs=pl.BlockSpec((ROW_BLOCK, H2), lambda i: (i, 0)),
        out_shape=jax.ShapeDtypeStruct((HASH_BIN, H2), jnp.float32),
    )(table, W1, b1, W2, b2)


@functools.lru_cache(maxsize=None)
def _make_sc_gather():
    mesh = plsc.VectorSubcoreMesh(core_axis_name="c", subcore_axis_name="s")

    @functools.partial(
        pl.kernel,
        out_type=jax.ShapeDtypeStruct((TOTAL, H2), jnp.float32),
        mesh=mesh,
        scratch_types=[
            pltpu.VMEM((NCHUNK, CHUNK), jnp.int32),
            pltpu.VMEM((CHUNK, H2), jnp.float32),
            pltpu.SemaphoreType.DMA,
        ],
        compiler_params=pltpu.CompilerParams(use_tc_tiling_on_sc=False),
    )
    def _sc_gather(t2_hbm, idx_hbm, out_hbm, idx_v, rows_v, sem):
        wid = lax.axis_index("s") * 2 + lax.axis_index("c")
        pltpu.sync_copy(idx_hbm.at[wid], idx_v)
        base = wid * ROWS_PER_W

        def body(j, carry):
            pltpu.async_copy(t2_hbm.at[idx_v.at[j]], rows_v, sem).wait()
            pltpu.sync_copy(rows_v, out_hbm.at[pl.ds(base + j * CHUNK, CHUNK)])
            return carry

        lax.fori_loop(0, NCHUNK, body, 0)

    return _sc_gather


def kernel(inputs, table, W1, b1, W2, b2):
    t2 = _table_mlp(table, W1.astype(jnp.float32), b1.reshape(1, H1),
                    W2.astype(jnp.float32), b2.reshape(1, H2))
    idx = jnp.mod(inputs, HASH_BIN).reshape(NW, NCHUNK, CHUNK)
    out = _make_sc_gather()(t2, idx)
    return out.reshape(BATCH, FIELDS * H2)


# speedup vs baseline: 6.1450x; 1.1692x over previous
import functools

import jax
import jax.numpy as jnp
from jax import lax
from jax.experimental import pallas as pl
from jax.experimental.pallas import tpu as pltpu
from jax.experimental.pallas import tpu_sc as plsc

HASH_BIN = 100000
EMB_DIM = 32
H1 = 32
H2 = 64
BATCH = 16384
FIELDS = 26

TOTAL = BATCH * FIELDS
NW = 32
ROWS_PER_W = TOTAL // NW
CHUNK = 128
NCHUNK = ROWS_PER_W // CHUNK

ROW_BLOCK = 2000
N_BLOCKS = HASH_BIN // ROW_BLOCK


def _mlp_body(t_ref, w1_ref, b1_ref, w2_ref, b2_ref, o_ref):
    h = jnp.dot(t_ref[...], w1_ref[...], preferred_element_type=jnp.float32)
    h = jnp.maximum(h + b1_ref[...], 0.0)
    o = jnp.dot(h, w2_ref[...], preferred_element_type=jnp.float32)
    o_ref[...] = jnp.maximum(o + b2_ref[...], 0.0)


def _table_mlp(table, W1, b1, W2, b2):
    return pl.pallas_call(
        _mlp_body,
        grid=(N_BLOCKS,),
        in_specs=[
            pl.BlockSpec((ROW_BLOCK, EMB_DIM), lambda i: (i, 0)),
            pl.BlockSpec((EMB_DIM, H1), lambda i: (0, 0)),
            pl.BlockSpec((1, H1), lambda i: (0, 0)),
            pl.BlockSpec((H1, H2), lambda i: (0, 0)),
            pl.BlockSpec((1, H2), lambda i: (0, 0)),
        ],
        out_specs=pl.BlockSpec((ROW_BLOCK, H2), lambda i: (i, 0)),
        out_shape=jax.ShapeDtypeStruct((HASH_BIN, H2), jnp.float32),
    )(table, W1, b1, W2, b2)


@functools.lru_cache(maxsize=None)
def _make_sc_gather():
    mesh = plsc.VectorSubcoreMesh(core_axis_name="c", subcore_axis_name="s")

    @functools.partial(
        pl.kernel,
        out_type=jax.ShapeDtypeStruct((TOTAL, H2), jnp.float32),
        mesh=mesh,
        scratch_types=[
            pltpu.VMEM((NCHUNK, CHUNK), jnp.int32),
            pltpu.VMEM((CHUNK, H2), jnp.float32),
            pltpu.VMEM((CHUNK, H2), jnp.float32),
            pltpu.SemaphoreType.DMA,
            pltpu.SemaphoreType.DMA,
            pltpu.SemaphoreType.DMA,
            pltpu.SemaphoreType.DMA,
        ],
        compiler_params=pltpu.CompilerParams(use_tc_tiling_on_sc=False),
    )
    def _sc_gather(t2_hbm, idx_hbm, out_hbm, idx_v, rows0, rows1,
                   gsem0, gsem1, wsem0, wsem1):
        wid = lax.axis_index("s") * 2 + lax.axis_index("c")
        pltpu.sync_copy(idx_hbm.at[wid], idx_v)
        base = wid * ROWS_PER_W
        rows = (rows0, rows1)
        gsem = (gsem0, gsem1)
        wsem = (wsem0, wsem1)

        def g_start(j, b):
            pltpu.async_copy(t2_hbm.at[idx_v.at[j]], rows[b], gsem[b])

        def g_wait(b):
            pltpu.make_async_copy(t2_hbm.at[idx_v.at[0]], rows[b],
                                  gsem[b]).wait()

        def w_start(j, b):
            pltpu.async_copy(rows[b],
                             out_hbm.at[pl.ds(base + j * CHUNK, CHUNK)],
                             wsem[b])

        def w_wait(b):
            pltpu.make_async_copy(rows[b], out_hbm.at[pl.ds(base, CHUNK)],
                                  wsem[b]).wait()

        g_start(0, 0)

        def body(i, carry):
            for b in (0, 1):
                j = 2 * i + b
                nb = 1 - b

                @pl.when(j >= 1)
                def _():
                    w_wait(nb)

                g_start(j + 1, nb)
                g_wait(b)
                w_start(j, b)
            return carry

        lax.fori_loop(0, NCHUNK // 2 - 1, body, 0)

        w_wait(1)
        g_start(NCHUNK - 1, 1)
        g_wait(0)
        w_start(NCHUNK - 2, 0)
        g_wait(1)
        w_start(NCHUNK - 1, 1)
        w_wait(0)
        w_wait(1)

    return _sc_gather


def kernel(inputs, table, W1, b1, W2, b2):
    t2 = _table_mlp(table, W1.astype(jnp.float32), b1.reshape(1, H1),
                    W2.astype(jnp.float32), b2.reshape(1, H2))
    idx = jnp.mod(inputs, HASH_BIN).reshape(NW, NCHUNK, CHUNK)
    out = _make_sc_gather()(t2, idx)
    return out.reshape(BATCH, FIELDS * H2)
